# Initial kernel scaffold; baseline (speedup 1.0000x reference)
import jax
import jax.numpy as jnp
from jax.experimental import pallas as pl


def kernel(x, edge_index, edge_attr, params):
    raise NotImplementedError("write your pallas kernel here")


# speedup vs baseline: 25.6684x; 25.6684x over previous
import functools

import jax
import jax.numpy as jnp
from jax import lax
from jax.experimental import pallas as pl
from jax.experimental.pallas import tpu as pltpu
from jax.experimental.pallas import tpu_sc as plsc

NN = 320
EDGES = 10240
_SC_WORKERS = 16
_EW = EDGES // _SC_WORKERS
_NCH = _EW // 64
_MSLAB = NN * NN // _SC_WORKERS


def _sc_a_body(row_hbm, col_hbm, ew_hbm, a_out, deg_out,
               row_v, col_v, ew_v, idx_b, zer_v,
               deg_sh, a_sh):
    wid = lax.axis_index("s")

    zv = jnp.zeros((16,), jnp.float32)

    def _zb(i, carry):
        zer_v[pl.ds(i * 16, 16)] = zv
        return carry

    lax.fori_loop(0, _MSLAB // 16, _zb, 0)
    pltpu.sync_copy(zer_v, a_sh.at[pl.ds(wid * _MSLAB, _MSLAB)])

    @pl.when(wid == 0)
    def _():
        pltpu.sync_copy(zer_v.at[pl.ds(0, NN)], deg_sh)

    base = wid * _EW
    pltpu.sync_copy(row_hbm.at[pl.ds(base, _EW)], row_v)
    pltpu.sync_copy(col_hbm.at[pl.ds(base, _EW)], col_v)
    pltpu.sync_copy(ew_hbm.at[pl.ds(base, _EW)], ew_v)

    plsc.subcore_barrier()

    for j in range(_NCH):
        for k in range(4):
            s = j * 64 + k * 16
            idx_b[j, pl.ds(k * 16, 16)] = row_v[pl.ds(s, 16)]
    for j in range(_NCH):
        pltpu.sync_copy(ew_v.at[pl.ds(j * 64, 64)],
                        deg_sh.at[idx_b.at[j]], add=True)

    for j in range(_NCH):
        for k in range(4):
            s = j * 64 + k * 16
            r = row_v[pl.ds(s, 16)]
            c = col_v[pl.ds(s, 16)]
            idx_b[j, pl.ds(k * 16, 16)] = c * NN + r
    for j in range(_NCH):
        pltpu.sync_copy(ew_v.at[pl.ds(j * 64, 64)],
                        a_sh.at[idx_b.at[j]], add=True)

    plsc.subcore_barrier()

    pltpu.sync_copy(a_sh.at[pl.ds(wid * _MSLAB, _MSLAB)],
                    a_out.at[pl.ds(wid * _MSLAB, _MSLAB)])

    @pl.when(wid == 0)
    def _():
        pltpu.sync_copy(deg_sh, deg_out)


def _norm_body(a_ref, deg_ref, o_ref):
    deg = deg_ref[0]
    dis = jnp.where(deg > 0.0,
                    lax.rsqrt(jnp.where(deg > 0.0, deg, 1.0)), 0.0)
    o_ref[...] = -(dis[:, None] * a_ref[...] * dis[None, :])


def _build_m(row, col, ew):
    mesh = plsc.VectorSubcoreMesh(core_axis_name="c", subcore_axis_name="s",
                                  num_cores=1)
    f = pl.kernel(
        _sc_a_body,
        out_type=(jax.ShapeDtypeStruct((NN * NN,), jnp.float32),
                  jax.ShapeDtypeStruct((NN,), jnp.float32)),
        mesh=mesh,
        scratch_types=[
            pltpu.VMEM((_EW,), jnp.int32),
            pltpu.VMEM((_EW,), jnp.int32),
            pltpu.VMEM((_EW,), jnp.float32),
            pltpu.VMEM((_NCH, 64), jnp.int32),
            pltpu.VMEM((_MSLAB,), jnp.float32),
            pltpu.VMEM_SHARED((NN,), jnp.float32),
            pltpu.VMEM_SHARED((NN * NN,), jnp.float32),
        ],
    )
    a_flat, deg = f(row, col, ew)
    return pl.pallas_call(
        _norm_body,
        in_specs=[pl.BlockSpec((NN, NN), lambda: (0, 0)),
                  pl.BlockSpec((1, NN), lambda: (0, 0))],
        out_specs=pl.BlockSpec((NN, NN), lambda: (0, 0)),
        out_shape=jax.ShapeDtypeStruct((NN, NN), jnp.float32),
    )(a_flat.reshape(NN, NN), deg.reshape(1, NN))


def _pick_tb(t1):
    for tb in (16, 13, 12, 10, 8, 7, 6, 5, 4):
        if t1 % tb == 0:
            return tb
    return t1


def _tconv(x, w1, w2, w3, b_cat, bn=None, fc=None):
    B, T, _, Ci = x.shape
    kt, _, Co = w1.shape
    T1 = T - kt + 1
    TB = _pick_tb(T1)
    Cout = 3 if fc is not None else Co

    def body(*refs):
        if fc is not None:
            x_ref, w1_ref, w2_ref, w3_ref, b_ref, bn_ref, fcw_ref, fcb_ref, o_ref = refs
        elif bn is not None:
            x_ref, w1_ref, w2_ref, w3_ref, b_ref, bn_ref, o_ref = refs
        else:
            x_ref, w1_ref, w2_ref, w3_ref, b_ref, o_ref = refs
        t0 = pl.program_id(1) * TB
        p = jnp.zeros((TB * NN, Co), jnp.float32)
        q = jnp.zeros((TB * NN, Co), jnp.float32)
        r = jnp.zeros((TB * NN, Co), jnp.float32)
        for k in range(kt):
            xk = x_ref[0, pl.ds(t0 + k, TB)].reshape(TB * NN, Ci)
            p = p + jnp.dot(xk, w1_ref[k], preferred_element_type=jnp.float32)
            q = q + jnp.dot(xk, w2_ref[k], preferred_element_type=jnp.float32)
            r = r + jnp.dot(xk, w3_ref[k], preferred_element_type=jnp.float32)
        p = p + b_ref[0][None, :]
        q = q + b_ref[1][None, :]
        r = r + b_ref[2][None, :]
        sig = 1.0 / (1.0 + jnp.exp(-q))
        h = jnp.maximum(p * sig + r, 0.0)
        if bn is not None:
            h = h.reshape(TB, NN, Co)
            h = h * bn_ref[0][None, :, None] + bn_ref[1][None, :, None]
            h = h.reshape(TB * NN, Co)
        if fc is not None:
            h = jnp.dot(h, fcw_ref[...], preferred_element_type=jnp.float32)
            h = h + fcb_ref[0][None, :]
        o_ref[0] = h.reshape(TB, NN, Cout)

    in_specs = [
        pl.BlockSpec((1, T, NN, Ci), lambda b, t: (b, 0, 0, 0)),
        pl.BlockSpec((kt, Ci, Co), lambda b, t: (0, 0, 0)),
        pl.BlockSpec((kt, Ci, Co), lambda b, t: (0, 0, 0)),
        pl.BlockSpec((kt, Ci, Co), lambda b, t: (0, 0, 0)),
        pl.BlockSpec((3, Co), lambda b, t: (0, 0)),
    ]
    args = [x, w1, w2, w3, b_cat]
    if bn is not None:
        in_specs.append(pl.BlockSpec((2, NN), lambda b, t: (0, 0)))
        args.append(bn)
    if fc is not None:
        fcw, fcb = fc
        in_specs.append(pl.BlockSpec(fcw.shape, lambda b, t: (0, 0)))
        in_specs.append(pl.BlockSpec((1, 3), lambda b, t: (0, 0)))
        args.extend([fcw, fcb.reshape(1, 3)])
    return pl.pallas_call(
        body,
        grid=(B, T1 // TB),
        in_specs=in_specs,
        out_specs=pl.BlockSpec((1, TB, NN, Cout), lambda b, t: (b, t, 0, 0)),
        out_shape=jax.ShapeDtypeStruct((B, T1, NN, Cout), jnp.float32),
    )(*args)


def _cheb(v, m, w, b):
    BT, _, C = v.shape
    K = w.shape[0]

    def body(v_ref, m_ref, w_ref, b_ref, o_ref):
        x0 = v_ref[0]
        mm = m_ref[...]
        out = jnp.dot(x0, w_ref[0], preferred_element_type=jnp.float32)
        x1 = jnp.dot(mm, x0, preferred_element_type=jnp.float32, precision=lax.Precision.HIGHEST)
        out = out + jnp.dot(x1, w_ref[1], preferred_element_type=jnp.float32)
        tm2, tm1 = x0, x1
        for k in range(2, K):
            xk = 2.0 * jnp.dot(mm, tm1, preferred_element_type=jnp.float32, precision=lax.Precision.HIGHEST) - tm2
            out = out + jnp.dot(xk, w_ref[k], preferred_element_type=jnp.float32)
            tm2, tm1 = tm1, xk
        o_ref[0] = jnp.maximum(out + b_ref[0][None, :], 0.0)

    return pl.pallas_call(
        body,
        grid=(BT,),
        in_specs=[
            pl.BlockSpec((1, NN, C), lambda i: (i, 0, 0)),
            pl.BlockSpec((NN, NN), lambda i: (0, 0)),
            pl.BlockSpec((K, C, C), lambda i: (0, 0, 0)),
            pl.BlockSpec((1, C), lambda i: (0, 0)),
        ],
        out_specs=pl.BlockSpec((1, NN, C), lambda i: (i, 0, 0)),
        out_shape=jax.ShapeDtypeStruct((BT, NN, C), jnp.float32),
    )(v, m, w, b.reshape(1, C))


def _prep_w(w):
    return jnp.transpose(w[:, :, 0, :], (2, 1, 0))


def kernel(x, edge_index, edge_attr, params):
    row = edge_index[0].astype(jnp.int32)
    col = edge_index[1].astype(jnp.int32)
    m = _build_m(row, col, edge_attr)

    h = x
    n_blocks = 3
    for i in range(n_blocks):
        p = params[f"block{i + 1}"]
        w1, w2, w3 = (_prep_w(p[f"tc1_w{j}"]) for j in (1, 2, 3))
        b_cat = jnp.stack([p["tc1_b1"], p["tc1_b2"], p["tc1_b3"]])
        t0 = _tconv(h, w1, w2, w3, b_cat)
        B, T1, _, ch = t0.shape
        g = _cheb(t0.reshape(B * T1, NN, ch), m, p["cheb_W"], p["cheb_b"])
        g = g.reshape(B, T1, NN, ch)
        w1, w2, w3 = (_prep_w(p[f"tc2_w{j}"]) for j in (1, 2, 3))
        b_cat = jnp.stack([p["tc2_b1"], p["tc2_b2"], p["tc2_b3"]])
        scale = p["bn_w"] * lax.rsqrt(p["bn_var"] + 1e-5)
        shift = p["bn_b"] - p["bn_mean"] * scale
        bn = jnp.stack([scale, shift])
        fc = (params["fc_w"], params["fc_b"]) if i == n_blocks - 1 else None
        h = _tconv(g, w1, w2, w3, b_cat, bn=bn, fc=fc)
    return h
